# baseline (speedup 1.0000x reference)
import functools

import jax
import jax.numpy as jnp
from jax import lax
from jax.experimental import pallas as pl
from jax.experimental.pallas import tpu as pltpu
from jax.experimental.pallas import tpu_sc as plsc

H = 64
HP = 128
CHUNK = 40
LANES = 16
NBUF = 6
AHEAD = 3


@functools.partial(jax.jit, static_argnums=(3, 4))
def _emb_lookup(x_flat, emb_pad, pos_table, n_batch, seq):
    info = plsc.get_sparse_core_info()
    nw = info.num_cores * info.num_subcores
    seq_per_w = n_batch // nw
    cpseq = seq // CHUNK
    n_chunks = seq_per_w * cpseq
    rows_per_w = seq_per_w * seq

    mesh = plsc.VectorSubcoreMesh(core_axis_name="c", subcore_axis_name="s")

    @functools.partial(
        pl.kernel,
        mesh=mesh,
        out_type=jax.ShapeDtypeStruct((n_batch, seq, H), jnp.float32),
        scratch_types=[
            pltpu.VMEM((rows_per_w,), jnp.int32),
            pltpu.VMEM((NBUF, CHUNK, HP), jnp.float32),
            pltpu.VMEM((NBUF, CHUNK, H), jnp.float32),
            pltpu.VMEM((seq, H), jnp.float32),
            pltpu.SemaphoreType.DMA((NBUF,)),
            pltpu.SemaphoreType.DMA((NBUF,)),
        ],
    )
    def body(x_hbm, emb_hbm, pos_hbm, out_hbm, idx_v, g_v, rows_v, pos_v,
             gsem, osem):
        num_cores = info.num_cores
        wid = lax.axis_index("s") * num_cores + lax.axis_index("c")
        seq0 = wid * seq_per_w

        pltpu.sync_copy(x_hbm.at[pl.ds(wid * rows_per_w, rows_per_w)], idx_v)
        pltpu.sync_copy(pos_hbm, pos_v)

        def gather_copy(c):
            b = lax.rem(c, NBUF)
            return pltpu.make_async_copy(
                emb_hbm.at[idx_v.at[pl.ds(
                    pl.multiple_of(c * CHUNK, CHUNK), CHUNK)]],
                g_v.at[b], gsem.at[b])

        def out_copy(c):
            b = lax.rem(c, NBUF)
            phase = pl.multiple_of(lax.rem(c, cpseq) * CHUNK, 8)
            return pltpu.make_async_copy(
                rows_v.at[b],
                out_hbm.at[seq0 + c // cpseq, pl.ds(phase, CHUNK), :],
                osem.at[b])

        for d in range(AHEAD):
            gather_copy(d).start()

        def do_chunk(c, carry):
            b = lax.rem(c, NBUF)
            gather_copy(c).wait()

            @pl.when(c >= NBUF)
            def _wait_prev_out():
                out_copy(c - NBUF).wait()

            phase = lax.rem(c, cpseq) * CHUNK

            @plsc.parallel_loop(0, CHUNK, step=1, unroll=4)
            def add_row(r):
                for j in range(H // LANES):
                    rows_v[b, r, pl.ds(j * LANES, LANES)] = (
                        g_v[b, r, pl.ds(j * LANES, LANES)]
                        + pos_v[phase + r, pl.ds(j * LANES, LANES)]
                    )

            out_copy(c).start()

            @pl.when(c + AHEAD < n_chunks)
            def _gather_prefetch():
                gather_copy(c + AHEAD).start()

            return carry

        lax.fori_loop(0, n_chunks, do_chunk, 0)

        for c in range(n_chunks - NBUF, n_chunks):
            out_copy(c).wait()

    return body(x_flat, emb_pad, pos_table)


def kernel(x, emb_table, pos_table):
    b, s = x.shape
    x_flat = x.reshape(-1).astype(jnp.int32)
    emb_pad = jnp.pad(emb_table, ((0, 0), (0, HP - H)))
    return _emb_lookup(x_flat, emb_pad, pos_table, b, s)


# speedup vs baseline: 1.0382x; 1.0382x over previous
import functools

import jax
import jax.numpy as jnp
from jax import lax
from jax.experimental import pallas as pl
from jax.experimental.pallas import tpu as pltpu
from jax.experimental.pallas import tpu_sc as plsc

H = 64
HP = 128
PIECES = (96, 104)
PMAX = 104
LANES = 16
NBUF = 3
AHEAD = 2


@functools.partial(jax.jit, static_argnums=(3, 4))
def _emb_lookup(x_flat, emb_pad, pos_flat, n_batch, seq):
    info = plsc.get_sparse_core_info()
    nw = info.num_cores * info.num_subcores
    seq_per_w = n_batch // nw
    rows_per_w = seq_per_w * seq
    n_pieces = 2 * seq_per_w
    phases = (0, PIECES[0])

    mesh = plsc.VectorSubcoreMesh(core_axis_name="c", subcore_axis_name="s")

    @functools.partial(
        pl.kernel,
        mesh=mesh,
        out_type=jax.ShapeDtypeStruct((n_batch, seq, H), jnp.float32),
        scratch_types=[
            pltpu.VMEM((rows_per_w,), jnp.int32),
            pltpu.VMEM((NBUF, PMAX, HP), jnp.float32),
            pltpu.VMEM((NBUF, PMAX, H), jnp.float32),
            pltpu.VMEM((seq * H,), jnp.float32),
            pltpu.SemaphoreType.DMA((NBUF,)),
            pltpu.SemaphoreType.DMA((NBUF,)),
        ],
    )
    def body(x_hbm, emb_hbm, pos_hbm, out_hbm, idx_v, g_v, rows_v, pos_v,
             gsem, osem):
        num_cores = info.num_cores
        wid = lax.axis_index("s") * num_cores + lax.axis_index("c")
        seq0 = wid * seq_per_w

        pltpu.sync_copy(x_hbm.at[pl.ds(wid * rows_per_w, rows_per_w)], idx_v)
        pltpu.sync_copy(pos_hbm, pos_v)

        def gather_copy(s, p):
            c = 2 * s + p
            b = lax.rem(c, NBUF)
            off = pl.multiple_of(s * seq + phases[p], 8)
            return pltpu.make_async_copy(
                emb_hbm.at[idx_v.at[pl.ds(off, PIECES[p])]],
                g_v.at[b, pl.ds(0, PIECES[p]), :], gsem.at[b])

        def out_copy(s, p):
            c = 2 * s + p
            b = lax.rem(c, NBUF)
            return pltpu.make_async_copy(
                rows_v.at[b, pl.ds(0, PIECES[p]), :],
                out_hbm.at[seq0 + s, pl.ds(phases[p], PIECES[p]), :],
                osem.at[b])

        gather_copy(0, 0).start()
        gather_copy(0, 1).start()

        def do_piece(s, p):
            c = 2 * s + p
            b = lax.rem(c, NBUF)
            gather_copy(s, p).wait()

            @pl.when(c >= NBUF)
            def _wait_prev_out():
                out_copy(s - 2 + p, 1 - p).wait()

            @plsc.parallel_loop(0, PIECES[p], step=1, unroll=4)
            def add_row(r):
                pr = (phases[p] + r) * H
                for j in range(H // LANES):
                    rows_v[b, r, pl.ds(j * LANES, LANES)] = (
                        g_v[b, r, pl.ds(j * LANES, LANES)]
                        + pos_v[pl.ds(pr + j * LANES, LANES)]
                    )

            out_copy(s, p).start()

            @pl.when(c + AHEAD < n_pieces)
            def _gather_prefetch():
                gather_copy(s + 1, p).start()

        def do_seq(s, carry):
            do_piece(s, 0)
            do_piece(s, 1)
            return carry

        lax.fori_loop(0, seq_per_w, do_seq, 0)

        for k in range(NBUF):
            c = n_pieces - NBUF + k
            out_copy(c // 2, c % 2).wait()

    return body(x_flat, emb_pad, pos_flat)


def kernel(x, emb_table, pos_table):
    b, s = x.shape
    x_flat = x.reshape(-1).astype(jnp.int32)
    emb_pad = jnp.pad(emb_table, ((0, 0), (0, HP - H)))
    pos_flat = pos_table.reshape(-1)
    return _emb_lookup(x_flat, emb_pad, pos_flat, b, s)


# speedup vs baseline: 1.2583x; 1.2120x over previous
import functools

import jax
import jax.numpy as jnp
from jax import lax
from jax.experimental import pallas as pl
from jax.experimental.pallas import tpu as pltpu
from jax.experimental.pallas import tpu_sc as plsc

H = 64
HP = 128
CHUNK = 128
LANES = 16
NBUF = 3
AHEAD = 2
IDXN = 4


@functools.partial(jax.jit, static_argnums=(3, 4))
def _emb_lookup(x_flat, emb_pad, pos_table, n_rows, seq):
    info = plsc.get_sparse_core_info()
    nw = info.num_cores * info.num_subcores
    rows_per_w = n_rows // nw
    n_chunks = rows_per_w // CHUNK

    mesh = plsc.VectorSubcoreMesh(core_axis_name="c", subcore_axis_name="s")

    @functools.partial(
        pl.kernel,
        mesh=mesh,
        out_type=jax.ShapeDtypeStruct((n_rows, H), jnp.float32),
        scratch_types=[
            pltpu.VMEM((IDXN * CHUNK,), jnp.int32),
            pltpu.VMEM((NBUF, CHUNK, HP), jnp.float32),
            pltpu.VMEM((NBUF, CHUNK, H), jnp.float32),
            pltpu.VMEM((seq, H), jnp.float32),
            pltpu.SemaphoreType.DMA((NBUF,)),
            pltpu.SemaphoreType.DMA((NBUF,)),
            pltpu.SemaphoreType.DMA((IDXN,)),
        ],
    )
    def body(x_hbm, emb_hbm, pos_hbm, out_hbm, idx_v, g_v, rows_v, pos_v,
             gsem, osem, isem):
        num_cores = info.num_cores
        wid = lax.axis_index("s") * num_cores + lax.axis_index("c")
        row0 = wid * rows_per_w

        pltpu.sync_copy(pos_hbm, pos_v)

        def idx_copy(c):
            k = lax.rem(c, IDXN)
            return pltpu.make_async_copy(
                x_hbm.at[pl.ds(row0 + c * CHUNK, CHUNK)],
                idx_v.at[pl.ds(pl.multiple_of(k * CHUNK, CHUNK), CHUNK)],
                isem.at[k])

        def gather_copy(c):
            k = lax.rem(c, IDXN)
            b = lax.rem(c, NBUF)
            return pltpu.make_async_copy(
                emb_hbm.at[idx_v.at[pl.ds(pl.multiple_of(k * CHUNK, CHUNK),
                                          CHUNK)]],
                g_v.at[b], gsem.at[b])

        def out_copy(c):
            b = lax.rem(c, NBUF)
            return pltpu.make_async_copy(
                rows_v.at[b],
                out_hbm.at[pl.ds(row0 + c * CHUNK, CHUNK), :],
                osem.at[b])

        for k in range(IDXN):
            idx_copy(k).start()
        for d in range(AHEAD):
            idx_copy(d).wait()
            gather_copy(d).start()

        def do_chunk(c, carry):
            b = lax.rem(c, NBUF)
            gather_copy(c).wait()

            @pl.when(c >= NBUF)
            def _wait_prev_out():
                out_copy(c - NBUF).wait()

            q = lax.rem(c * CHUNK, seq)

            @plsc.parallel_loop(0, CHUNK, step=1, unroll=4)
            def add_row(r):
                s_pos = lax.rem(q + r, seq)
                for j in range(H // LANES):
                    rows_v[b, r, pl.ds(j * LANES, LANES)] = (
                        g_v[b, r, pl.ds(j * LANES, LANES)]
                        + pos_v[s_pos, pl.ds(j * LANES, LANES)]
                    )

            out_copy(c).start()

            @pl.when(c + IDXN < n_chunks)
            def _idx_prefetch():
                idx_copy(c + IDXN).start()

            @pl.when(c + AHEAD < n_chunks)
            def _gather_prefetch():
                idx_copy(c + AHEAD).wait()
                gather_copy(c + AHEAD).start()

            return carry

        lax.fori_loop(0, n_chunks, do_chunk, 0)

        for c in range(n_chunks - NBUF, n_chunks):
            out_copy(c).wait()

    return body(x_flat, emb_pad, pos_table)


def kernel(x, emb_table, pos_table):
    b, s = x.shape
    x_flat = x.reshape(-1).astype(jnp.int32)
    emb_pad = jnp.pad(emb_table, ((0, 0), (0, HP - H)))
    out = _emb_lookup(x_flat, emb_pad, pos_table, b * s, s)
    return out.reshape(b, s, H)
